# baseline (speedup 1.0000x reference)
import functools

import jax
import jax.numpy as jnp
from jax import lax
from jax.experimental import pallas as pl
from jax.experimental.pallas import tpu as pltpu
from jax.experimental.pallas import tpu_sc as plsc

N = 50000
K = 27
EPS = 1e-5

NW = 32
CH = 128
TOTP = 1437696
NCH = TOTP // (NW * CH)
NP = TOTP // K

R = 400
NT = N // R


TROWS = N + 16
TSL = TROWS // 16
KBUF = 3


@functools.partial(jax.jit, static_argnames=("cin",))
def _sc_gather(table, idx3, *, cin):
  mesh = plsc.VectorSubcoreMesh(core_axis_name="c", subcore_axis_name="s")

  @functools.partial(
      pl.kernel,
      mesh=mesh,
      out_type=jax.ShapeDtypeStruct((TOTP, cin), table.dtype),
      compiler_params=pltpu.CompilerParams(use_tc_tiling_on_sc=False),
      scratch_types=[
          pltpu.VMEM((2, KBUF, CH), jnp.int32),
          pltpu.VMEM((KBUF, CH, cin), table.dtype),
          pltpu.VMEM_SHARED((TROWS, cin), table.dtype),
          pltpu.SemaphoreType.DMA,
          pltpu.SemaphoreType.DMA,
      ],
  )
  def gather_kernel(table_hbm, idx_hbm, out_hbm, idx_v, rows_v, shared, sem,
                    isem):
    sid = lax.axis_index("s")
    wid = lax.axis_index("c") * 16 + sid
    base = wid * NCH * CH
    pltpu.sync_copy(table_hbm.at[pl.ds(sid * TSL, TSL)],
                    shared.at[pl.ds(sid * TSL, TSL)])
    cpi0 = pltpu.async_copy(idx_hbm.at[wid].at[pl.ds(0, KBUF)],
                            idx_v.at[0], isem)
    plsc.subcore_barrier()

    nsteps = NCH // KBUF

    def body(t, carry):
      j0 = t * KBUF
      sl = t % 2
      pltpu.make_async_copy(idx_hbm.at[wid].at[pl.ds(j0, KBUF)],
                            idx_v.at[sl], isem).wait()

      @pl.when(t + 1 < nsteps)
      def _prefetch():
        pltpu.async_copy(idx_hbm.at[wid].at[pl.ds(j0 + KBUF, KBUF)],
                         idx_v.at[1 - sl], isem)

      cps = [
          pltpu.async_copy(shared.at[idx_v.at[sl, b]], rows_v.at[b], sem)
          for b in range(KBUF)
      ]
      for b in range(KBUF):
        cps[b].wait()
        pltpu.sync_copy(rows_v.at[b],
                        out_hbm.at[pl.ds(base + (j0 + b) * CH, CH)])
      return carry

    del cpi0
    lax.fori_loop(0, nsteps, body, 0)

  return gather_kernel(table, idx3)


def _conv(gs, ws, res=None, want_stats=False):
  ng = len(gs)
  cout = ws[0].shape[1]

  def body(*refs):
    nin = 2 * ng + (0 if res is None else 1)
    orefs = refs[nin:]
    x = jnp.dot(refs[0][...], refs[ng][...],
                preferred_element_type=jnp.float32)
    for a in range(1, ng):
      x = x + jnp.dot(refs[a][...], refs[ng + a][...],
                      preferred_element_type=jnp.float32)
    if res is not None:
      x = x + refs[2 * ng][...]
    orefs[0][...] = x
    if want_stats:
      s = jnp.stack([jnp.sum(x, axis=0), jnp.sum(x * x, axis=0)])
      i = pl.program_id(0)

      @pl.when(i == 0)
      def _init():
        orefs[1][...] = s

      @pl.when(i > 0)
      def _acc():
        orefs[1][...] = orefs[1][...] + s

  in_specs = [pl.BlockSpec((g.shape[1], cout) if j else (R, g.shape[1]),
                           (lambda i: (0, 0)) if j else (lambda i: (i, 0)))
              for j in (0, 1) for g in gs]
  operands = list(gs) + list(ws)
  if res is not None:
    in_specs.append(pl.BlockSpec((R, cout), lambda i: (i, 0)))
    operands.append(res)
  out_shape = [jax.ShapeDtypeStruct((N, cout), jnp.float32)]
  out_specs = [pl.BlockSpec((R, cout), lambda i: (i, 0))]
  if want_stats:
    out_shape.append(jax.ShapeDtypeStruct((2, cout), jnp.float32))
    out_specs.append(pl.BlockSpec((2, cout), lambda i: (0, 0)))
  out = pl.pallas_call(
      body,
      grid=(NT,),
      in_specs=in_specs,
      out_specs=out_specs,
      out_shape=out_shape,
  )(*operands)
  return out if want_stats else out[0]


def _bnrelu(x, stats, g, b, ws=None):
  c = x.shape[1]
  g2 = g.reshape(1, c)
  b2 = b.reshape(1, c)

  def body(x_ref, s_ref, g_ref, b_ref, *rest):
    mu = s_ref[0:1, :] * (1.0 / N)
    var = s_ref[1:2, :] * (1.0 / N) - mu * mu
    scale = g_ref[...] * lax.rsqrt(var + EPS)
    h = jnp.maximum((x_ref[...] - mu) * scale + b_ref[...], 0.0)
    if ws is None:
      rest[-1][...] = h.astype(jnp.bfloat16)
    else:
      w_ref = rest[0]
      rest[1][...] = h.astype(jnp.bfloat16)
      rest[2][...] = jnp.dot(h, w_ref[...],
                             preferred_element_type=jnp.float32)

  in_specs = [
      pl.BlockSpec((R, c), lambda i: (i, 0)),
      pl.BlockSpec((2, c), lambda i: (0, 0)),
      pl.BlockSpec((1, c), lambda i: (0, 0)),
      pl.BlockSpec((1, c), lambda i: (0, 0)),
  ]
  operands = [x, stats, g2, b2]
  out_shape = [jax.ShapeDtypeStruct((N, c), jnp.bfloat16)]
  out_specs = [pl.BlockSpec((R, c), lambda i: (i, 0))]
  if ws is not None:
    cs = ws.shape[1]
    in_specs.append(pl.BlockSpec((c, cs), lambda i: (0, 0)))
    operands.append(ws)
    out_shape.append(jax.ShapeDtypeStruct((N, cs), jnp.float32))
    out_specs.append(pl.BlockSpec((R, cs), lambda i: (i, 0)))
  out = pl.pallas_call(
      body,
      grid=(NT,),
      in_specs=in_specs,
      out_specs=out_specs,
      out_shape=out_shape,
  )(*operands)
  return out[0] if ws is None else out


def _build_src_idx(pin, pout, st):
  e = jnp.arange(pin.shape[0], dtype=jnp.int32)
  st32 = st.astype(jnp.int32)
  seg = jnp.sum(e[:, None] >= st32[None, :], axis=1).astype(jnp.int32) - 1
  dest = pout.astype(jnp.int32) * K + seg
  idx = jnp.full((TOTP,), N, dtype=jnp.int32).at[dest].set(
      pin.astype(jnp.int32), unique_indices=True)
  return idx.reshape(NW, NCH, CH)


def _pad_table(h):
  c = h.shape[1]
  return jnp.zeros((TROWS, c), jnp.bfloat16).at[:N].set(h)


def kernel(coords, feats, pin1, pout1, st1, ct1, pin2, pout2, st2, ct2,
           pin4, pout4, st4, ct4, stem_W, s0_g1, s0_b1, s0_W1, s0_g2, s0_b2,
           s0_W2, s1_g1, s1_b1, s1_W1, s1_g2, s1_b2, s1_W2, s1_Ws):
  idx1 = _build_src_idx(pin1, pout1, st1)
  idx2 = _build_src_idx(pin2, pout2, st2)
  idx4 = _build_src_idx(pin4, pout4, st4)

  h_tab = jnp.zeros((TROWS, 16), jnp.bfloat16).at[:N, :4].set(
      feats.astype(jnp.bfloat16))
  w0 = jnp.zeros((K, 16, 32), jnp.bfloat16).at[:, :4, :].set(
      stem_W.astype(jnp.bfloat16))
  g1 = _sc_gather(h_tab, idx1, cin=16).reshape(NP, K * 16)
  x0, st_x0 = _conv([g1], [w0.reshape(K * 16, 32)], want_stats=True)

  h0 = _bnrelu(x0, st_x0, s0_g1, s0_b1)
  g2 = _sc_gather(_pad_table(h0), idx2, cin=32).reshape(NP, K * 32)
  t0, st_t0 = _conv([g2], [s0_W1.astype(jnp.bfloat16).reshape(K * 32, 32)], want_stats=True)
  h1 = _bnrelu(t0, st_t0, s0_g2, s0_b2)
  g3 = _sc_gather(_pad_table(h1), idx2, cin=32).reshape(NP, K * 32)
  x1, st_x1 = _conv([g3], [s0_W2.astype(jnp.bfloat16).reshape(K * 32, 32)], res=x0,
                    want_stats=True)

  pre, sc = _bnrelu(x1, st_x1, s1_g1, s1_b1, ws=s1_Ws)
  g4 = _sc_gather(_pad_table(pre), idx4, cin=32).reshape(NP, K * 32)
  t1, st_t1 = _conv([g4], [s1_W1.astype(jnp.bfloat16).reshape(K * 32, 64)], want_stats=True)
  h2 = _bnrelu(t1, st_t1, s1_g2, s1_b2)
  g5 = _sc_gather(_pad_table(h2), idx4, cin=64).reshape(NP, K * 64)
  x2 = _conv([g5], [s1_W2.astype(jnp.bfloat16).reshape(K * 64, 64)], res=sc)

  return (x0, x1, x2)


# speedup vs baseline: 1.1086x; 1.1086x over previous
import functools

import jax
import jax.numpy as jnp
from jax import lax
from jax.experimental import pallas as pl
from jax.experimental.pallas import tpu as pltpu
from jax.experimental.pallas import tpu_sc as plsc

N = 50000
K = 27
EPS = 1e-5

NW = 32
CH = 384
TOTP = 1437696
NCH = TOTP // (NW * CH)
NP = TOTP // K

R = 400
NT = N // R


TROWS = N + 16
TSL = TROWS // 16
KBUF = 2


@functools.partial(jax.jit, static_argnames=("cin",))
def _sc_gather(table, idx3, *, cin):
  mesh = plsc.VectorSubcoreMesh(core_axis_name="c", subcore_axis_name="s")

  @functools.partial(
      pl.kernel,
      mesh=mesh,
      out_type=jax.ShapeDtypeStruct((TOTP, cin), jnp.float32),
      compiler_params=pltpu.CompilerParams(use_tc_tiling_on_sc=False),
      scratch_types=[
          pltpu.VMEM((2, KBUF, CH), jnp.int32),
          pltpu.VMEM((KBUF, CH, cin), jnp.float32),
          pltpu.VMEM_SHARED((TROWS, cin), jnp.float32),
          pltpu.SemaphoreType.DMA,
          pltpu.SemaphoreType.DMA,
          pltpu.SemaphoreType.DMA,
      ],
  )
  def gather_kernel(table_hbm, idx_hbm, out_hbm, idx_v, rows_v, shared, sem,
                    isem, wsem):
    sid = lax.axis_index("s")
    wid = lax.axis_index("c") * 16 + sid
    base = wid * NCH * CH
    pltpu.sync_copy(table_hbm.at[pl.ds(sid * TSL, TSL)],
                    shared.at[pl.ds(sid * TSL, TSL)])
    cpi0 = pltpu.async_copy(idx_hbm.at[wid].at[pl.ds(0, KBUF)],
                            idx_v.at[0], isem)
    plsc.subcore_barrier()

    nsteps = NCH // KBUF

    def body(t, carry):
      j0 = t * KBUF
      sl = t % 2
      pltpu.make_async_copy(idx_hbm.at[wid].at[pl.ds(j0, KBUF)],
                            idx_v.at[sl], isem).wait()

      @pl.when(t + 1 < nsteps)
      def _prefetch():
        pltpu.async_copy(idx_hbm.at[wid].at[pl.ds(j0 + KBUF, KBUF)],
                         idx_v.at[1 - sl], isem)

      @pl.when(t > 0)
      def _drain():
        for b in range(KBUF):
          pltpu.make_async_copy(
              rows_v.at[b], out_hbm.at[pl.ds(base, CH)], wsem).wait()

      cps = [
          pltpu.async_copy(shared.at[idx_v.at[sl, b]], rows_v.at[b], sem)
          for b in range(KBUF)
      ]
      for b in range(KBUF):
        cps[b].wait()
        pltpu.async_copy(rows_v.at[b],
                         out_hbm.at[pl.ds(base + (j0 + b) * CH, CH)], wsem)
      return carry

    del cpi0
    lax.fori_loop(0, nsteps, body, 0)

    for b in range(KBUF):
      pltpu.make_async_copy(rows_v.at[b], out_hbm.at[pl.ds(base, CH)],
                            wsem).wait()
    j = NCH - 1
    pltpu.sync_copy(idx_hbm.at[wid].at[pl.ds(j, 1)], idx_v.at[0, 0:1])
    pltpu.async_copy(shared.at[idx_v.at[0, 0]], rows_v.at[0], sem).wait()
    pltpu.sync_copy(rows_v.at[0], out_hbm.at[pl.ds(base + j * CH, CH)])

  return gather_kernel(table, idx3)


def _conv(gs, ws, res=None, want_stats=False):
  ng = len(gs)
  cout = ws[0].shape[1]

  def body(*refs):
    nin = 2 * ng + (0 if res is None else 1)
    orefs = refs[nin:]
    x = jnp.dot(refs[0][...], refs[ng][...],
                preferred_element_type=jnp.float32)
    for a in range(1, ng):
      x = x + jnp.dot(refs[a][...], refs[ng + a][...],
                      preferred_element_type=jnp.float32)
    if res is not None:
      x = x + refs[2 * ng][...]
    orefs[0][...] = x
    if want_stats:
      s = jnp.stack([jnp.sum(x, axis=0), jnp.sum(x * x, axis=0)])
      i = pl.program_id(0)

      @pl.when(i == 0)
      def _init():
        orefs[1][...] = s

      @pl.when(i > 0)
      def _acc():
        orefs[1][...] = orefs[1][...] + s

  in_specs = [pl.BlockSpec((g.shape[1], cout) if j else (R, g.shape[1]),
                           (lambda i: (0, 0)) if j else (lambda i: (i, 0)))
              for j in (0, 1) for g in gs]
  operands = list(gs) + list(ws)
  if res is not None:
    in_specs.append(pl.BlockSpec((R, cout), lambda i: (i, 0)))
    operands.append(res)
  out_shape = [jax.ShapeDtypeStruct((N, cout), jnp.float32)]
  out_specs = [pl.BlockSpec((R, cout), lambda i: (i, 0))]
  if want_stats:
    out_shape.append(jax.ShapeDtypeStruct((2, cout), jnp.float32))
    out_specs.append(pl.BlockSpec((2, cout), lambda i: (0, 0)))
  out = pl.pallas_call(
      body,
      grid=(NT,),
      in_specs=in_specs,
      out_specs=out_specs,
      out_shape=out_shape,
  )(*operands)
  return out if want_stats else out[0]


def _bnrelu(x, stats, g, b, ws=None):
  c = x.shape[1]
  g2 = g.reshape(1, c)
  b2 = b.reshape(1, c)

  def body(x_ref, s_ref, g_ref, b_ref, *rest):
    mu = s_ref[0:1, :] * (1.0 / N)
    var = s_ref[1:2, :] * (1.0 / N) - mu * mu
    scale = g_ref[...] * lax.rsqrt(var + EPS)
    h = jnp.maximum((x_ref[...] - mu) * scale + b_ref[...], 0.0)
    if ws is None:
      rest[-1][...] = h
    else:
      w_ref = rest[0]
      rest[1][...] = h
      rest[2][...] = jnp.dot(h, w_ref[...],
                             preferred_element_type=jnp.float32)

  in_specs = [
      pl.BlockSpec((R, c), lambda i: (i, 0)),
      pl.BlockSpec((2, c), lambda i: (0, 0)),
      pl.BlockSpec((1, c), lambda i: (0, 0)),
      pl.BlockSpec((1, c), lambda i: (0, 0)),
  ]
  operands = [x, stats, g2, b2]
  out_shape = [jax.ShapeDtypeStruct((N, c), jnp.float32)]
  out_specs = [pl.BlockSpec((R, c), lambda i: (i, 0))]
  if ws is not None:
    cs = ws.shape[1]
    in_specs.append(pl.BlockSpec((c, cs), lambda i: (0, 0)))
    operands.append(ws)
    out_shape.append(jax.ShapeDtypeStruct((N, cs), jnp.float32))
    out_specs.append(pl.BlockSpec((R, cs), lambda i: (i, 0)))
  out = pl.pallas_call(
      body,
      grid=(NT,),
      in_specs=in_specs,
      out_specs=out_specs,
      out_shape=out_shape,
  )(*operands)
  return out[0] if ws is None else out


def _build_src_idx(pin, pout, st):
  e = jnp.arange(pin.shape[0], dtype=jnp.int32)
  st32 = st.astype(jnp.int32)
  seg = jnp.sum(e[:, None] >= st32[None, :], axis=1).astype(jnp.int32) - 1
  dest = pout.astype(jnp.int32) * K + seg
  idx = jnp.full((TOTP,), N, dtype=jnp.int32).at[dest].set(
      pin.astype(jnp.int32), unique_indices=True)
  return idx.reshape(NW, NCH, CH)


def _pad_table(h):
  c = h.shape[1]
  return jnp.zeros((TROWS, c), jnp.float32).at[:N].set(h)


def kernel(coords, feats, pin1, pout1, st1, ct1, pin2, pout2, st2, ct2,
           pin4, pout4, st4, ct4, stem_W, s0_g1, s0_b1, s0_W1, s0_g2, s0_b2,
           s0_W2, s1_g1, s1_b1, s1_W1, s1_g2, s1_b2, s1_W2, s1_Ws):
  idx1 = _build_src_idx(pin1, pout1, st1)
  idx2 = _build_src_idx(pin2, pout2, st2)
  idx4 = _build_src_idx(pin4, pout4, st4)

  h_tab = jnp.zeros((TROWS, 16), jnp.float32).at[:N, :4].set(feats)
  w0 = jnp.zeros((K, 16, 32), jnp.float32).at[:, :4, :].set(stem_W)
  g1 = _sc_gather(h_tab, idx1, cin=16).reshape(NP, K * 16)
  x0, st_x0 = _conv([g1], [w0.reshape(K * 16, 32)], want_stats=True)

  h0 = _bnrelu(x0, st_x0, s0_g1, s0_b1)
  g2 = _sc_gather(_pad_table(h0), idx2, cin=32).reshape(NP, K * 32)
  t0, st_t0 = _conv([g2], [s0_W1.reshape(K * 32, 32)], want_stats=True)
  h1 = _bnrelu(t0, st_t0, s0_g2, s0_b2)
  g3 = _sc_gather(_pad_table(h1), idx2, cin=32).reshape(NP, K * 32)
  x1, st_x1 = _conv([g3], [s0_W2.reshape(K * 32, 32)], res=x0,
                    want_stats=True)

  pre, sc = _bnrelu(x1, st_x1, s1_g1, s1_b1, ws=s1_Ws)
  g4 = _sc_gather(_pad_table(pre), idx4, cin=32).reshape(NP, K * 32)
  t1, st_t1 = _conv([g4], [s1_W1.reshape(K * 32, 64)], want_stats=True)
  h2 = _bnrelu(t1, st_t1, s1_g2, s1_b2)
  g5a = _sc_gather(_pad_table(h2[:, :32]), idx4, cin=32).reshape(NP, K * 32)
  g5b = _sc_gather(_pad_table(h2[:, 32:]), idx4, cin=32).reshape(NP, K * 32)
  x2 = _conv([g5a, g5b],
             [s1_W2[:, :32, :].reshape(K * 32, 64),
              s1_W2[:, 32:, :].reshape(K * 32, 64)], res=sc)

  return (x0, x1, x2)


# speedup vs baseline: 1.8768x; 1.6929x over previous
import functools

import jax
import jax.numpy as jnp
from jax import lax
from jax.experimental import pallas as pl
from jax.experimental.pallas import tpu as pltpu
from jax.experimental.pallas import tpu_sc as plsc

N = 50000
K = 27
EPS = 1e-5

NW = 32
CH = 384
TOTP = 1437696
NCH = TOTP // (NW * CH)
NP = TOTP // K

R = 400
NT = N // R


TROWS = N + 16
TSL = TROWS // 16
KBUF = 2


@functools.partial(jax.jit, static_argnames=("cin",))
def _sc_gather(table, idx3, *, cin):
  mesh = plsc.VectorSubcoreMesh(core_axis_name="c", subcore_axis_name="s")

  @functools.partial(
      pl.kernel,
      mesh=mesh,
      out_type=jax.ShapeDtypeStruct((TOTP, cin), jnp.float32),
      compiler_params=pltpu.CompilerParams(use_tc_tiling_on_sc=False),
      scratch_types=[
          pltpu.VMEM((2, KBUF, CH), jnp.int32),
          pltpu.VMEM((KBUF, CH, cin), jnp.float32),
          pltpu.VMEM_SHARED((TROWS, cin), jnp.float32),
          pltpu.SemaphoreType.DMA,
          pltpu.SemaphoreType.DMA,
          pltpu.SemaphoreType.DMA,
      ],
  )
  def gather_kernel(table_hbm, idx_hbm, out_hbm, idx_v, rows_v, shared, sem,
                    isem, wsem):
    sid = lax.axis_index("s")
    wid = lax.axis_index("c") * 16 + sid
    base = wid * NCH * CH
    pltpu.sync_copy(table_hbm.at[pl.ds(sid * TSL, TSL)],
                    shared.at[pl.ds(sid * TSL, TSL)])
    cpi0 = pltpu.async_copy(idx_hbm.at[wid].at[pl.ds(0, KBUF)],
                            idx_v.at[0], isem)
    plsc.subcore_barrier()

    nsteps = NCH // KBUF

    def body(t, carry):
      j0 = t * KBUF
      sl = t % 2
      pltpu.make_async_copy(idx_hbm.at[wid].at[pl.ds(j0, KBUF)],
                            idx_v.at[sl], isem).wait()

      @pl.when(t + 1 < nsteps)
      def _prefetch():
        pltpu.async_copy(idx_hbm.at[wid].at[pl.ds(j0 + KBUF, KBUF)],
                         idx_v.at[1 - sl], isem)

      @pl.when(t > 0)
      def _drain():
        for b in range(KBUF):
          pltpu.make_async_copy(
              rows_v.at[b], out_hbm.at[pl.ds(base, CH)], wsem).wait()

      cps = [
          pltpu.async_copy(shared.at[idx_v.at[sl, b]], rows_v.at[b], sem)
          for b in range(KBUF)
      ]
      for b in range(KBUF):
        cps[b].wait()
        pltpu.async_copy(rows_v.at[b],
                         out_hbm.at[pl.ds(base + (j0 + b) * CH, CH)], wsem)
      return carry

    del cpi0
    lax.fori_loop(0, nsteps, body, 0)

    for b in range(KBUF):
      pltpu.make_async_copy(rows_v.at[b], out_hbm.at[pl.ds(base, CH)],
                            wsem).wait()
    j = NCH - 1
    pltpu.sync_copy(idx_hbm.at[wid].at[pl.ds(j, 1)], idx_v.at[0, 0:1])
    pltpu.async_copy(shared.at[idx_v.at[0, 0]], rows_v.at[0], sem).wait()
    pltpu.sync_copy(rows_v.at[0], out_hbm.at[pl.ds(base + j * CH, CH)])

  return gather_kernel(table, idx3)


def _conv(gs, ws, res=None, want_stats=False):
  ng = len(gs)
  cout = ws[0].shape[1]

  def body(*refs):
    nin = 2 * ng + (0 if res is None else 1)
    orefs = refs[nin:]
    x = jnp.dot(refs[0][...], refs[ng][...],
                preferred_element_type=jnp.float32)
    for a in range(1, ng):
      x = x + jnp.dot(refs[a][...], refs[ng + a][...],
                      preferred_element_type=jnp.float32)
    if res is not None:
      x = x + refs[2 * ng][...]
    orefs[0][...] = x
    if want_stats:
      s = jnp.stack([jnp.sum(x, axis=0), jnp.sum(x * x, axis=0)])
      i = pl.program_id(0)

      @pl.when(i == 0)
      def _init():
        orefs[1][...] = s

      @pl.when(i > 0)
      def _acc():
        orefs[1][...] = orefs[1][...] + s

  in_specs = [pl.BlockSpec((g.shape[1], cout) if j else (R, g.shape[1]),
                           (lambda i: (0, 0)) if j else (lambda i: (i, 0)))
              for j in (0, 1) for g in gs]
  operands = list(gs) + list(ws)
  if res is not None:
    in_specs.append(pl.BlockSpec((R, cout), lambda i: (i, 0)))
    operands.append(res)
  out_shape = [jax.ShapeDtypeStruct((N, cout), jnp.float32)]
  out_specs = [pl.BlockSpec((R, cout), lambda i: (i, 0))]
  if want_stats:
    out_shape.append(jax.ShapeDtypeStruct((2, cout), jnp.float32))
    out_specs.append(pl.BlockSpec((2, cout), lambda i: (0, 0)))
  out = pl.pallas_call(
      body,
      grid=(NT,),
      in_specs=in_specs,
      out_specs=out_specs,
      out_shape=out_shape,
  )(*operands)
  return out if want_stats else out[0]


def _bnrelu(x, stats, g, b, ws=None):
  c = x.shape[1]
  g2 = g.reshape(1, c)
  b2 = b.reshape(1, c)

  def body(x_ref, s_ref, g_ref, b_ref, *rest):
    mu = s_ref[0:1, :] * (1.0 / N)
    var = s_ref[1:2, :] * (1.0 / N) - mu * mu
    scale = g_ref[...] * lax.rsqrt(var + EPS)
    h = jnp.maximum((x_ref[...] - mu) * scale + b_ref[...], 0.0)
    if ws is None:
      rest[-1][...] = h
    else:
      w_ref = rest[0]
      rest[1][...] = h
      rest[2][...] = jnp.dot(h, w_ref[...],
                             preferred_element_type=jnp.float32)

  in_specs = [
      pl.BlockSpec((R, c), lambda i: (i, 0)),
      pl.BlockSpec((2, c), lambda i: (0, 0)),
      pl.BlockSpec((1, c), lambda i: (0, 0)),
      pl.BlockSpec((1, c), lambda i: (0, 0)),
  ]
  operands = [x, stats, g2, b2]
  out_shape = [jax.ShapeDtypeStruct((N, c), jnp.float32)]
  out_specs = [pl.BlockSpec((R, c), lambda i: (i, 0))]
  if ws is not None:
    cs = ws.shape[1]
    in_specs.append(pl.BlockSpec((c, cs), lambda i: (0, 0)))
    operands.append(ws)
    out_shape.append(jax.ShapeDtypeStruct((N, cs), jnp.float32))
    out_specs.append(pl.BlockSpec((R, cs), lambda i: (i, 0)))
  out = pl.pallas_call(
      body,
      grid=(NT,),
      in_specs=in_specs,
      out_specs=out_specs,
      out_shape=out_shape,
  )(*operands)
  return out[0] if ws is None else out


CHE = 1664
TSEG = TOTP // 16


def _build_src_idx(pin, pout, st):
  e = jnp.arange(pin.shape[0], dtype=jnp.int32)
  st32 = st.astype(jnp.int32)
  seg = jnp.sum(e[:, None] >= st32[None, :], axis=1).astype(jnp.int32) - 1
  dest = pout.astype(jnp.int32) * K + seg
  ep = -(-pin.shape[0] // (16 * CHE)) * (16 * CHE)
  pad = ep - pin.shape[0]
  dest3 = jnp.concatenate(
      [dest, jnp.full((pad,), TOTP, jnp.int32)]).reshape(16, -1, CHE)
  pin3 = jnp.concatenate(
      [pin.astype(jnp.int32), jnp.full((pad,), N, jnp.int32)]
  ).reshape(16, -1, CHE)
  fill = jnp.full((CHE,), N, jnp.int32)
  nech = ep // (16 * CHE)

  mesh = plsc.VectorSubcoreMesh(core_axis_name="c", subcore_axis_name="s",
                                num_cores=1)

  @functools.partial(
      pl.kernel,
      mesh=mesh,
      out_type=jax.ShapeDtypeStruct((TOTP,), jnp.int32),
      compiler_params=pltpu.CompilerParams(use_tc_tiling_on_sc=False),
      scratch_types=[
          pltpu.VMEM((2, CHE), jnp.int32),
          pltpu.VMEM((2, CHE), jnp.int32),
          pltpu.VMEM((CHE,), jnp.int32),
          pltpu.VMEM_SHARED((TOTP + 8,), jnp.int32),
          pltpu.SemaphoreType.DMA,
          pltpu.SemaphoreType.DMA,
      ],
  )
  def build_kernel(pin_hbm, dest_hbm, fill_hbm, out_hbm, pv, dv, fv,
                   table_sp, sem, wsem):
    sid = lax.axis_index("s")
    pltpu.sync_copy(fill_hbm, fv)

    def mset(m, carry):
      pltpu.async_copy(
          fv, table_sp.at[pl.ds(sid * TSEG + m * CHE, CHE)], wsem)
      return carry

    lax.fori_loop(0, TSEG // CHE, mset, 0)

    def mdrain(m, carry):
      pltpu.make_async_copy(
          fv, table_sp.at[pl.ds(sid * TSEG, CHE)], wsem).wait()
      return carry

    lax.fori_loop(0, TSEG // CHE, mdrain, 0)
    plsc.subcore_barrier()

    def ebody(j, carry):
      cpp = pltpu.async_copy(pin_hbm.at[sid].at[j], pv.at[0], sem)
      cpd = pltpu.async_copy(dest_hbm.at[sid].at[j], dv.at[0], sem)
      cpp.wait()
      cpd.wait()
      pltpu.sync_copy(pv.at[0], table_sp.at[dv.at[0]])
      return carry

    lax.fori_loop(0, nech, ebody, 0)
    plsc.subcore_barrier()
    pltpu.sync_copy(table_sp.at[pl.ds(sid * TSEG, TSEG)],
                    out_hbm.at[pl.ds(sid * TSEG, TSEG)])

  return build_kernel(pin3, dest3, fill).reshape(NW, NCH, CH)


def _pad_table(h):
  c = h.shape[1]
  return jnp.zeros((TROWS, c), jnp.float32).at[:N].set(h)


def kernel(coords, feats, pin1, pout1, st1, ct1, pin2, pout2, st2, ct2,
           pin4, pout4, st4, ct4, stem_W, s0_g1, s0_b1, s0_W1, s0_g2, s0_b2,
           s0_W2, s1_g1, s1_b1, s1_W1, s1_g2, s1_b2, s1_W2, s1_Ws):
  idx1 = _build_src_idx(pin1, pout1, st1)
  idx2 = _build_src_idx(pin2, pout2, st2)
  idx4 = _build_src_idx(pin4, pout4, st4)

  h_tab = jnp.zeros((TROWS, 16), jnp.float32).at[:N, :4].set(feats)
  w0 = jnp.zeros((K, 16, 32), jnp.float32).at[:, :4, :].set(stem_W)
  g1 = _sc_gather(h_tab, idx1, cin=16).reshape(NP, K * 16)
  x0, st_x0 = _conv([g1], [w0.reshape(K * 16, 32)], want_stats=True)

  h0 = _bnrelu(x0, st_x0, s0_g1, s0_b1)
  g2 = _sc_gather(_pad_table(h0), idx2, cin=32).reshape(NP, K * 32)
  t0, st_t0 = _conv([g2], [s0_W1.reshape(K * 32, 32)], want_stats=True)
  h1 = _bnrelu(t0, st_t0, s0_g2, s0_b2)
  g3 = _sc_gather(_pad_table(h1), idx2, cin=32).reshape(NP, K * 32)
  x1, st_x1 = _conv([g3], [s0_W2.reshape(K * 32, 32)], res=x0,
                    want_stats=True)

  pre, sc = _bnrelu(x1, st_x1, s1_g1, s1_b1, ws=s1_Ws)
  g4 = _sc_gather(_pad_table(pre), idx4, cin=32).reshape(NP, K * 32)
  t1, st_t1 = _conv([g4], [s1_W1.reshape(K * 32, 64)], want_stats=True)
  h2 = _bnrelu(t1, st_t1, s1_g2, s1_b2)
  g5a = _sc_gather(_pad_table(h2[:, :32]), idx4, cin=32).reshape(NP, K * 32)
  g5b = _sc_gather(_pad_table(h2[:, 32:]), idx4, cin=32).reshape(NP, K * 32)
  x2 = _conv([g5a, g5b],
             [s1_W2[:, :32, :].reshape(K * 32, 64),
              s1_W2[:, 32:, :].reshape(K * 32, 64)], res=sc)

  return (x0, x1, x2)


# speedup vs baseline: 1.9124x; 1.0189x over previous
import functools

import jax
import jax.numpy as jnp
from jax import lax
from jax.experimental import pallas as pl
from jax.experimental.pallas import tpu as pltpu
from jax.experimental.pallas import tpu_sc as plsc

N = 50000
K = 27
EPS = 1e-5

NW = 32
CH = 384
TOTP = 1437696
NCH = TOTP // (NW * CH)
NP = TOTP // K

R = 400
NT = N // R


TROWS = 50400
TSL = TROWS // 16
NTB = TROWS // R
KBUF = 2


@functools.partial(jax.jit, static_argnames=("cin",))
def _sc_gather(table, idx3, *, cin):
  mesh = plsc.VectorSubcoreMesh(core_axis_name="c", subcore_axis_name="s")

  @functools.partial(
      pl.kernel,
      mesh=mesh,
      out_type=jax.ShapeDtypeStruct((TOTP, cin), jnp.float32),
      compiler_params=pltpu.CompilerParams(use_tc_tiling_on_sc=False),
      scratch_types=[
          pltpu.VMEM((2, KBUF, CH), jnp.int32),
          pltpu.VMEM((KBUF, CH, cin), jnp.float32),
          pltpu.VMEM_SHARED((TROWS, cin), jnp.float32),
          pltpu.SemaphoreType.DMA,
          pltpu.SemaphoreType.DMA,
          pltpu.SemaphoreType.DMA,
      ],
  )
  def gather_kernel(table_hbm, idx_hbm, out_hbm, idx_v, rows_v, shared, sem,
                    isem, wsem):
    sid = lax.axis_index("s")
    wid = lax.axis_index("c") * 16 + sid
    base = wid * NCH * CH
    pltpu.sync_copy(table_hbm.at[pl.ds(sid * TSL, TSL)],
                    shared.at[pl.ds(sid * TSL, TSL)])
    cpi0 = pltpu.async_copy(idx_hbm.at[wid].at[pl.ds(0, KBUF)],
                            idx_v.at[0], isem)
    plsc.subcore_barrier()

    nsteps = NCH // KBUF

    def body(t, carry):
      j0 = t * KBUF
      sl = t % 2
      pltpu.make_async_copy(idx_hbm.at[wid].at[pl.ds(j0, KBUF)],
                            idx_v.at[sl], isem).wait()

      @pl.when(t + 1 < nsteps)
      def _prefetch():
        pltpu.async_copy(idx_hbm.at[wid].at[pl.ds(j0 + KBUF, KBUF)],
                         idx_v.at[1 - sl], isem)

      @pl.when(t > 0)
      def _drain():
        for b in range(KBUF):
          pltpu.make_async_copy(
              rows_v.at[b], out_hbm.at[pl.ds(base, CH)], wsem).wait()

      cps = [
          pltpu.async_copy(shared.at[idx_v.at[sl, b]], rows_v.at[b], sem)
          for b in range(KBUF)
      ]
      for b in range(KBUF):
        cps[b].wait()
        pltpu.async_copy(rows_v.at[b],
                         out_hbm.at[pl.ds(base + (j0 + b) * CH, CH)], wsem)
      return carry

    del cpi0
    lax.fori_loop(0, nsteps, body, 0)

    for b in range(KBUF):
      pltpu.make_async_copy(rows_v.at[b], out_hbm.at[pl.ds(base, CH)],
                            wsem).wait()
    j = NCH - 1
    pltpu.sync_copy(idx_hbm.at[wid].at[pl.ds(j, 1)], idx_v.at[0, 0:1])
    pltpu.async_copy(shared.at[idx_v.at[0, 0]], rows_v.at[0], sem).wait()
    pltpu.sync_copy(rows_v.at[0], out_hbm.at[pl.ds(base + j * CH, CH)])

  return gather_kernel(table, idx3)


def _conv(gs, ws, res=None, want_stats=False):
  ng = len(gs)
  cout = ws[0].shape[1]

  def body(*refs):
    nin = 2 * ng + (0 if res is None else 1)
    orefs = refs[nin:]
    x = jnp.dot(refs[0][...], refs[ng][...],
                preferred_element_type=jnp.float32)
    for a in range(1, ng):
      x = x + jnp.dot(refs[a][...], refs[ng + a][...],
                      preferred_element_type=jnp.float32)
    if res is not None:
      x = x + refs[2 * ng][...]
    orefs[0][...] = x
    if want_stats:
      s = jnp.stack([jnp.sum(x, axis=0), jnp.sum(x * x, axis=0)])
      i = pl.program_id(0)

      @pl.when(i == 0)
      def _init():
        orefs[1][...] = s

      @pl.when(i > 0)
      def _acc():
        orefs[1][...] = orefs[1][...] + s

  in_specs = [pl.BlockSpec((g.shape[1], cout) if j else (R, g.shape[1]),
                           (lambda i: (0, 0)) if j else (lambda i: (i, 0)))
              for j in (0, 1) for g in gs]
  operands = list(gs) + list(ws)
  if res is not None:
    in_specs.append(pl.BlockSpec((R, cout), lambda i: (i, 0)))
    operands.append(res)
  out_shape = [jax.ShapeDtypeStruct((N, cout), jnp.float32)]
  out_specs = [pl.BlockSpec((R, cout), lambda i: (i, 0))]
  if want_stats:
    out_shape.append(jax.ShapeDtypeStruct((2, cout), jnp.float32))
    out_specs.append(pl.BlockSpec((2, cout), lambda i: (0, 0)))
  out = pl.pallas_call(
      body,
      grid=(NT,),
      in_specs=in_specs,
      out_specs=out_specs,
      out_shape=out_shape,
  )(*operands)
  return out if want_stats else out[0]


def _bnrelu(x, stats, g, b, ws=None, split=False):
  c = x.shape[1]
  g2 = g.reshape(1, c)
  b2 = b.reshape(1, c)

  def body(x_ref, s_ref, g_ref, b_ref, *rest):
    mu = s_ref[0:1, :] * (1.0 / N)
    var = s_ref[1:2, :] * (1.0 / N) - mu * mu
    scale = g_ref[...] * lax.rsqrt(var + EPS)
    h = jnp.maximum((x_ref[...] - mu) * scale + b_ref[...], 0.0)
    i = pl.program_id(0)
    row = i * R + lax.broadcasted_iota(jnp.int32, (R, 1), 0)
    h = jnp.where(row < N, h, 0.0)
    if split:
      rest[0][...] = h[:, :c // 2]
      rest[1][...] = h[:, c // 2:]
    elif ws is None:
      rest[0][...] = h
    else:
      rest[1][...] = h
      rest[2][...] = jnp.dot(h, rest[0][...],
                             preferred_element_type=jnp.float32)

  clamp = lambda i: (jnp.minimum(i, NT - 1), 0)
  in_specs = [
      pl.BlockSpec((R, c), clamp),
      pl.BlockSpec((2, c), lambda i: (0, 0)),
      pl.BlockSpec((1, c), lambda i: (0, 0)),
      pl.BlockSpec((1, c), lambda i: (0, 0)),
  ]
  operands = [x, stats, g2, b2]
  if split:
    out_shape = [jax.ShapeDtypeStruct((TROWS, c // 2), jnp.float32)] * 2
    out_specs = [pl.BlockSpec((R, c // 2), lambda i: (i, 0))] * 2
  else:
    out_shape = [jax.ShapeDtypeStruct((TROWS, c), jnp.float32)]
    out_specs = [pl.BlockSpec((R, c), lambda i: (i, 0))]
    if ws is not None:
      cs = ws.shape[1]
      in_specs.append(pl.BlockSpec((c, cs), lambda i: (0, 0)))
      operands.append(ws)
      out_shape.append(jax.ShapeDtypeStruct((TROWS, cs), jnp.float32))
      out_specs.append(pl.BlockSpec((R, cs), lambda i: (i, 0)))
  out = pl.pallas_call(
      body,
      grid=(NTB,),
      in_specs=in_specs,
      out_specs=out_specs,
      out_shape=out_shape,
  )(*operands)
  return out[0] if (ws is None and not split) else out


CHE = 1664
TSEG = TOTP // 16


def _build_src_idx(pin, pout, st):
  e = jnp.arange(pin.shape[0], dtype=jnp.int32)
  st32 = st.astype(jnp.int32)
  seg = jnp.sum(e[:, None] >= st32[None, :], axis=1).astype(jnp.int32) - 1
  dest = pout.astype(jnp.int32) * K + seg
  ep = -(-pin.shape[0] // (16 * CHE)) * (16 * CHE)
  pad = ep - pin.shape[0]
  dest3 = jnp.concatenate(
      [dest, jnp.full((pad,), TOTP, jnp.int32)]).reshape(16, -1, CHE)
  pin3 = jnp.concatenate(
      [pin.astype(jnp.int32), jnp.full((pad,), N, jnp.int32)]
  ).reshape(16, -1, CHE)
  fill = jnp.full((CHE,), N, jnp.int32)
  nech = ep // (16 * CHE)

  mesh = plsc.VectorSubcoreMesh(core_axis_name="c", subcore_axis_name="s",
                                num_cores=1)

  @functools.partial(
      pl.kernel,
      mesh=mesh,
      out_type=jax.ShapeDtypeStruct((TOTP,), jnp.int32),
      compiler_params=pltpu.CompilerParams(use_tc_tiling_on_sc=False),
      scratch_types=[
          pltpu.VMEM((2, CHE), jnp.int32),
          pltpu.VMEM((2, CHE), jnp.int32),
          pltpu.VMEM((CHE,), jnp.int32),
          pltpu.VMEM_SHARED((TOTP + 8,), jnp.int32),
          pltpu.SemaphoreType.DMA,
          pltpu.SemaphoreType.DMA,
      ],
  )
  def build_kernel(pin_hbm, dest_hbm, fill_hbm, out_hbm, pv, dv, fv,
                   table_sp, sem, wsem):
    sid = lax.axis_index("s")
    pltpu.sync_copy(fill_hbm, fv)

    def mset(m, carry):
      pltpu.async_copy(
          fv, table_sp.at[pl.ds(sid * TSEG + m * CHE, CHE)], wsem)
      return carry

    lax.fori_loop(0, TSEG // CHE, mset, 0)

    def mdrain(m, carry):
      pltpu.make_async_copy(
          fv, table_sp.at[pl.ds(sid * TSEG, CHE)], wsem).wait()
      return carry

    lax.fori_loop(0, TSEG // CHE, mdrain, 0)
    plsc.subcore_barrier()

    def ebody(j, carry):
      cpp = pltpu.async_copy(pin_hbm.at[sid].at[j], pv.at[0], sem)
      cpd = pltpu.async_copy(dest_hbm.at[sid].at[j], dv.at[0], sem)
      cpp.wait()
      cpd.wait()
      pltpu.sync_copy(pv.at[0], table_sp.at[dv.at[0]])
      return carry

    lax.fori_loop(0, nech, ebody, 0)
    plsc.subcore_barrier()
    pltpu.sync_copy(table_sp.at[pl.ds(sid * TSEG, TSEG)],
                    out_hbm.at[pl.ds(sid * TSEG, TSEG)])

  return build_kernel(pin3, dest3, fill).reshape(NW, NCH, CH)


def kernel(coords, feats, pin1, pout1, st1, ct1, pin2, pout2, st2, ct2,
           pin4, pout4, st4, ct4, stem_W, s0_g1, s0_b1, s0_W1, s0_g2, s0_b2,
           s0_W2, s1_g1, s1_b1, s1_W1, s1_g2, s1_b2, s1_W2, s1_Ws):
  idx1 = _build_src_idx(pin1, pout1, st1)
  idx2 = _build_src_idx(pin2, pout2, st2)
  idx4 = _build_src_idx(pin4, pout4, st4)

  h_tab = jnp.zeros((TROWS, 16), jnp.float32).at[:N, :4].set(feats)
  w0 = jnp.zeros((K, 16, 32), jnp.float32).at[:, :4, :].set(stem_W)
  g1 = _sc_gather(h_tab, idx1, cin=16).reshape(NP, K * 16)
  x0, st_x0 = _conv([g1], [w0.reshape(K * 16, 32)], want_stats=True)

  h0 = _bnrelu(x0, st_x0, s0_g1, s0_b1)
  g2 = _sc_gather(h0, idx2, cin=32).reshape(NP, K * 32)
  t0, st_t0 = _conv([g2], [s0_W1.reshape(K * 32, 32)], want_stats=True)
  h1 = _bnrelu(t0, st_t0, s0_g2, s0_b2)
  g3 = _sc_gather(h1, idx2, cin=32).reshape(NP, K * 32)
  x1, st_x1 = _conv([g3], [s0_W2.reshape(K * 32, 32)], res=x0,
                    want_stats=True)

  pre, sc = _bnrelu(x1, st_x1, s1_g1, s1_b1, ws=s1_Ws)
  g4 = _sc_gather(pre, idx4, cin=32).reshape(NP, K * 32)
  t1, st_t1 = _conv([g4], [s1_W1.reshape(K * 32, 64)], want_stats=True)
  h2a, h2b = _bnrelu(t1, st_t1, s1_g2, s1_b2, split=True)
  g5a = _sc_gather(h2a, idx4, cin=32).reshape(NP, K * 32)
  g5b = _sc_gather(h2b, idx4, cin=32).reshape(NP, K * 32)
  x2 = _conv([g5a, g5b],
             [s1_W2[:, :32, :].reshape(K * 32, 64),
              s1_W2[:, 32:, :].reshape(K * 32, 64)], res=sc)

  return (x0, x1, x2)
